# baseline (speedup 1.0000x reference)
import functools

import jax
import jax.numpy as jnp
from jax import lax
from jax.experimental import pallas as pl
from jax.experimental.pallas import tpu as pltpu
from jax.experimental.pallas import tpu_sc as plsc

NC = 2
NS = 16
HD = 128
IR = 128
CR = 128
BM = 1000


def _sc_segment_sums(table, didx, cidx, zacc, nr):
    ndb = didx.shape[3]
    ncb = cidx.shape[3]
    nx = (table.shape[0] - CR) // 2
    rpt = nr // NS
    mesh = plsc.VectorSubcoreMesh(core_axis_name="c", subcore_axis_name="s")

    @functools.partial(
        pl.kernel,
        mesh=mesh,
        out_type=jax.ShapeDtypeStruct((NC, 2, nr, HD), jnp.float32),
        scratch_types=[
            pltpu.VMEM((8, IR), jnp.int32),
            pltpu.VMEM((IR, HD), jnp.float32),
            pltpu.VMEM((IR, HD), jnp.float32),
            pltpu.VMEM_SHARED((CR, HD), jnp.float32),
            pltpu.VMEM_SHARED((CR, HD), jnp.float32),
            pltpu.VMEM_SHARED((nr, HD), jnp.float32),
            pltpu.SemaphoreType.DMA,
            pltpu.SemaphoreType.DMA,
        ],
    )
    def sc_kernel(table_hbm, didx_hbm, cidx_hbm, zacc_hbm, a_hbm,
                  idx_v, buf0, buf1, ident_v, zs, acc, sem0, sem1):
        c = lax.axis_index("c")
        s = lax.axis_index("s")
        row0 = s * rpt
        bufs = (buf0, buf1)
        sems = (sem0, sem1)
        @pl.when(s == 0)
        def _():
            pltpu.sync_copy(table_hbm.at[pl.ds(2 * nx, CR)], ident_v)
            pltpu.sync_copy(zacc_hbm.at[pl.ds(0, CR)], zs)
        plsc.subcore_barrier()

        def run_blocks(src_ref, iref, nblk, rel, local):
            @pl.loop(0, nblk)
            def _(t):
                pltpu.sync_copy(iref.at[c, rel, s, t], idx_v)
                if local:
                    for b in range(4):
                        pltpu.sync_copy(src_ref.at[idx_v.at[b]], buf0)
                        pltpu.sync_copy(buf0, acc.at[idx_v.at[4 + b]],
                                        add=True)
                    return
                d = pltpu.async_copy(src_ref.at[idx_v.at[0]], buf0, sem0)
                for b in range(4):
                    if b < 3:
                        nxt = pltpu.async_copy(
                            src_ref.at[idx_v.at[b + 1]],
                            bufs[(b + 1) % 2], sems[(b + 1) % 2])
                    d.wait()
                    pltpu.sync_copy(bufs[b % 2], acc.at[idx_v.at[4 + b]],
                                    add=True)
                    if b < 3:
                        d = nxt

        nz, rz = rpt // CR, rpt % CR
        for rel in range(2):
            for k in range(nz):
                pltpu.sync_copy(zs, acc.at[pl.ds(row0 + k * CR, CR)])
            if rz:
                pltpu.sync_copy(zs.at[pl.ds(0, rz)],
                                acc.at[pl.ds(row0 + nz * CR, rz)])
            plsc.subcore_barrier()
            run_blocks(table_hbm, didx_hbm, ndb, rel, local=False)
            run_blocks(ident_v, cidx_hbm, ncb, rel, local=True)
            plsc.subcore_barrier()
            pltpu.sync_copy(acc.at[pl.ds(row0, rpt)],
                            a_hbm.at[c, rel, pl.ds(row0, rpt)])
            plsc.subcore_barrier()

    return sc_kernel(table, didx, cidx, zacc)


def _tc_body(x_ref, a_ref, cnt_ref, w_ref, b_ref, ws_ref, bs_ref, o_ref):
    f32 = jnp.float32
    out = jnp.dot(x_ref[...], ws_ref[...], preferred_element_type=f32)
    out += bs_ref[...][None, :]
    for rel in range(2):
        for c in range(NC):
            out += jnp.dot(a_ref[c, rel],
                           w_ref[rel][c * HD:(c + 1) * HD, :],
                           preferred_element_type=f32)
        cnt = cnt_ref[0, rel, 0, 0] + cnt_ref[1, rel, 0, 0]
        out += cnt[:, None] * b_ref[rel][None, :]
    o_ref[...] = out


def _tc_combine(x, a, cnt, w, b, Ws, bs):
    n, d = x.shape
    dout = Ws.shape[1]
    bm = BM
    return pl.pallas_call(
        _tc_body,
        grid=(n // bm,),
        in_specs=[
            pl.BlockSpec((bm, d), lambda i: (i, 0)),
            pl.BlockSpec((NC, 2, bm, HD), lambda i: (0, 0, i, 0)),
            pl.BlockSpec((NC, 2, 1, 1, bm), lambda i: (0, 0, i, 0, 0)),
            pl.BlockSpec((2, d, dout), lambda i: (0, 0, 0)),
            pl.BlockSpec((2, dout), lambda i: (0, 0)),
            pl.BlockSpec((d, dout), lambda i: (0, 0)),
            pl.BlockSpec((dout,), lambda i: (0,)),
        ],
        out_specs=pl.BlockSpec((bm, dout), lambda i: (i, 0)),
        out_shape=jax.ShapeDtypeStruct((n, dout), jnp.float32),
    )(x, a, cnt, w, b, Ws, bs)


def kernel(x, edge_rel0, edge_rel1, edge_self, W0, b0, W1, b1, Ws, bs):
    n, d = x.shape
    e = edge_rel0.shape[1]
    xh = x.reshape(n, 2, HD).transpose(1, 0, 2).reshape(2 * n, HD)
    table = jnp.concatenate([xh, jnp.eye(CR, dtype=jnp.float32)], axis=0)

    n_pad = ((n // NS + 7) // 8 * 8) * NS
    if n_pad == n:
        n_pad += 8 * NS
    nr = n_pad + CR
    ept = e // NS
    blk = 4 * IR
    ept_pad = (ept + blk - 1) // blk * blk
    nb_all = ept_pad // blk

    def prep(edge):
        src = edge[0].reshape(NS, ept)
        dst = edge[1].reshape(NS, ept)
        pad = ept_pad - ept
        shp = (NS, nb_all, 4, IR)
        srcp = jnp.pad(src, ((0, 0), (0, pad))).reshape(shp)
        dstp = jnp.pad(dst, ((0, 0), (0, pad)),
                       constant_values=n_pad - 1).reshape(shp)
        csrc = jnp.pad(dst % CR, ((0, 0), (0, pad)),
                       constant_values=0).reshape(shp)
        cdst = jnp.pad(n_pad + dst // CR, ((0, 0), (0, pad)),
                       constant_values=n_pad - 1).reshape(shp)
        return srcp, dstp, csrc, cdst

    def pack(a_, b_):
        return jnp.concatenate([a_, b_], axis=2)

    s0, d0, cs0, cd0 = prep(edge_rel0)
    s1, d1, cs1, cd1 = prep(edge_rel1)
    didx = jnp.stack([
        jnp.stack([pack(s0, d0), pack(s1, d1)]),
        jnp.stack([pack(s0 + n, d0), pack(s1 + n, d1)]),
    ])
    h = nb_all // 2
    c0p, c1p = pack(cs0, cd0), pack(cs1, cd1)
    cidx = jnp.stack([
        jnp.stack([c0p[:, :h], c1p[:, :h]]),
        jnp.stack([c0p[:, h:], c1p[:, h:]]),
    ])
    zacc = jnp.zeros((CR, HD), jnp.float32)

    a = _sc_segment_sums(table, didx, cidx, zacc, nr)

    cnt = a[:, :, n_pad:, :].reshape(NC, 2, CR * HD)[:, :, :n]
    cnt = cnt.reshape(NC, 2, n // BM, 1, BM)
    w = jnp.stack([W0, W1])
    b = jnp.stack([b0, b1])
    return _tc_combine(x, a, cnt, w, b, Ws, bs)


# speedup vs baseline: 2.4111x; 2.4111x over previous
import functools

import jax
import jax.numpy as jnp
from jax import lax
from jax.experimental import pallas as pl
from jax.experimental.pallas import tpu as pltpu
from jax.experimental.pallas import tpu_sc as plsc

NC = 2
NS = 16
HD = 128
IR = 128
CR = 128
BM = 1000


def _sc_segment_sums(table, didx, cidx, zacc, nr):
    ndb = didx.shape[3]
    ncb = cidx.shape[3]
    nx = (table.shape[0] - CR) // 2
    rpt = nr // NS
    mesh = plsc.VectorSubcoreMesh(core_axis_name="c", subcore_axis_name="s")

    @functools.partial(
        pl.kernel,
        mesh=mesh,
        out_type=jax.ShapeDtypeStruct((NC, 2, nr, HD), jnp.float32),
        scratch_types=[
            pltpu.VMEM((8, IR), jnp.int32),
            pltpu.VMEM((IR, HD), jnp.float32),
            pltpu.VMEM((IR, HD), jnp.float32),
            pltpu.VMEM_SHARED((CR, HD), jnp.float32),
            pltpu.VMEM_SHARED((nr, HD), jnp.float32),
            pltpu.SemaphoreType.DMA,
            pltpu.SemaphoreType.DMA,
        ],
    )
    def sc_kernel(table_hbm, didx_hbm, cidx_hbm, zacc_hbm, a_hbm,
                  idx_v, buf0, buf1, ident_v, acc, sem0, sem1):
        c = lax.axis_index("c")
        s = lax.axis_index("s")
        row0 = s * rpt
        bufs = (buf0, buf1)
        sems = (sem0, sem1)
        @pl.when(s == 0)
        def _():
            pltpu.sync_copy(table_hbm.at[pl.ds(2 * nx, CR)], ident_v)
        plsc.subcore_barrier()

        def run_blocks(src_ref, iref, nblk, rel, local):
            @pl.loop(0, nblk)
            def _(t):
                pltpu.sync_copy(iref.at[c, rel, s, t], idx_v)
                if local:
                    for b in range(4):
                        pltpu.sync_copy(src_ref.at[idx_v.at[b]], buf0)
                        pltpu.sync_copy(buf0, acc.at[idx_v.at[4 + b]],
                                        add=True)
                    return
                d = pltpu.async_copy(src_ref.at[idx_v.at[0]], buf0, sem0)
                for b in range(4):
                    if b < 3:
                        nxt = pltpu.async_copy(
                            src_ref.at[idx_v.at[b + 1]],
                            bufs[(b + 1) % 2], sems[(b + 1) % 2])
                    d.wait()
                    pltpu.sync_copy(bufs[b % 2], acc.at[idx_v.at[4 + b]],
                                    add=True)
                    if b < 3:
                        d = nxt

        for rel in range(2):
            pltpu.sync_copy(zacc_hbm, acc.at[pl.ds(row0, rpt)])
            plsc.subcore_barrier()
            run_blocks(table_hbm, didx_hbm, ndb, rel, local=False)
            run_blocks(ident_v, cidx_hbm, ncb, rel, local=True)
            plsc.subcore_barrier()
            pltpu.sync_copy(acc.at[pl.ds(row0, rpt)],
                            a_hbm.at[c, rel, pl.ds(row0, rpt)])
            plsc.subcore_barrier()

    return sc_kernel(table, didx, cidx, zacc)


def _tc_body(x_ref, a_ref, cnt_ref, w_ref, b_ref, ws_ref, bs_ref, o_ref):
    f32 = jnp.float32
    out = jnp.dot(x_ref[...], ws_ref[...], preferred_element_type=f32)
    out += bs_ref[...][None, :]
    for rel in range(2):
        for c in range(NC):
            out += jnp.dot(a_ref[c, rel],
                           w_ref[rel][c * HD:(c + 1) * HD, :],
                           preferred_element_type=f32)
        cnt = cnt_ref[0, rel, 0, 0] + cnt_ref[1, rel, 0, 0]
        out += cnt[:, None] * b_ref[rel][None, :]
    o_ref[...] = out


def _tc_combine(x, a, cnt, w, b, Ws, bs):
    n, d = x.shape
    dout = Ws.shape[1]
    bm = BM
    return pl.pallas_call(
        _tc_body,
        grid=(n // bm,),
        in_specs=[
            pl.BlockSpec((bm, d), lambda i: (i, 0)),
            pl.BlockSpec((NC, 2, bm, HD), lambda i: (0, 0, i, 0)),
            pl.BlockSpec((NC, 2, 1, 1, bm), lambda i: (0, 0, i, 0, 0)),
            pl.BlockSpec((2, d, dout), lambda i: (0, 0, 0)),
            pl.BlockSpec((2, dout), lambda i: (0, 0)),
            pl.BlockSpec((d, dout), lambda i: (0, 0)),
            pl.BlockSpec((dout,), lambda i: (0,)),
        ],
        out_specs=pl.BlockSpec((bm, dout), lambda i: (i, 0)),
        out_shape=jax.ShapeDtypeStruct((n, dout), jnp.float32),
    )(x, a, cnt, w, b, Ws, bs)


def kernel(x, edge_rel0, edge_rel1, edge_self, W0, b0, W1, b1, Ws, bs):
    n, d = x.shape
    e = edge_rel0.shape[1]
    xh = x.reshape(n, 2, HD).transpose(1, 0, 2).reshape(2 * n, HD)
    table = jnp.concatenate([xh, jnp.eye(CR, dtype=jnp.float32)], axis=0)

    n_pad = ((n // NS + 7) // 8 * 8) * NS
    if n_pad == n:
        n_pad += 8 * NS
    nr = n_pad + CR
    ept = e // NS
    blk = 4 * IR
    ept_pad = (ept + blk - 1) // blk * blk
    nb_all = ept_pad // blk

    def prep(edge):
        src = edge[0].reshape(NS, ept)
        dst = edge[1].reshape(NS, ept)
        pad = ept_pad - ept
        shp = (NS, nb_all, 4, IR)
        srcp = jnp.pad(src, ((0, 0), (0, pad))).reshape(shp)
        dstp = jnp.pad(dst, ((0, 0), (0, pad)),
                       constant_values=n_pad - 1).reshape(shp)
        csrc = jnp.pad(dst % CR, ((0, 0), (0, pad)),
                       constant_values=0).reshape(shp)
        cdst = jnp.pad(n_pad + dst // CR, ((0, 0), (0, pad)),
                       constant_values=n_pad - 1).reshape(shp)
        return srcp, dstp, csrc, cdst

    def pack(a_, b_):
        return jnp.concatenate([a_, b_], axis=2)

    s0, d0, cs0, cd0 = prep(edge_rel0)
    s1, d1, cs1, cd1 = prep(edge_rel1)
    didx = jnp.stack([
        jnp.stack([pack(s0, d0), pack(s1, d1)]),
        jnp.stack([pack(s0 + n, d0), pack(s1 + n, d1)]),
    ])
    h = nb_all // 2
    c0p, c1p = pack(cs0, cd0), pack(cs1, cd1)
    cidx = jnp.stack([
        jnp.stack([c0p[:, :h], c1p[:, :h]]),
        jnp.stack([c0p[:, h:], c1p[:, h:]]),
    ])
    zacc = jnp.zeros((nr // NS, HD), jnp.float32)

    a = _sc_segment_sums(table, didx, cidx, zacc, nr)

    cnt = a[:, :, n_pad:, :].reshape(NC, 2, CR * HD)[:, :, :n]
    cnt = cnt.reshape(NC, 2, n // BM, 1, BM)
    w = jnp.stack([W0, W1])
    b = jnp.stack([b0, b1])
    return _tc_combine(x, a, cnt, w, b, Ws, bs)
